# baseline (speedup 1.0000x reference)
import functools

import jax
import jax.numpy as jnp
from jax.experimental import pallas as pl
from jax.experimental.pallas import tpu as pltpu

_F32 = jnp.float32
_BF16 = jnp.bfloat16


def _rup(x, m):
    return ((x + m - 1) // m) * m


def _conv_body(x_ref, w_ref, s_ref, b_ref, *rest, wo_real, relu, has_res):
    i_rest = 0
    if has_res:
        res_ref = rest[i_rest]
        i_rest += 1
    o_ref = rest[i_rest]

    bb, hi, wi, cin = x_ref.shape
    cout = w_ref.shape[3]
    ho = hi - 2
    wo8 = wi - 2
    m = bb * ho * wo8

    x = x_ref[...]
    acc = jnp.zeros((m, cout), _F32)
    for i in range(3):
        for j in range(3):
            win = jax.lax.slice(x, (0, i, j, 0),
                                (bb, i + ho, j + wo8, cin))
            acc = acc + jnp.dot(win.reshape(m, cin), w_ref[i, j],
                                preferred_element_type=_F32)

    out = acc * s_ref[...] + b_ref[...]
    out = out.reshape(bb, ho, wo8, cout)
    if has_res:
        out = out + res_ref[...].astype(_F32)
    if relu:
        out = jnp.maximum(out, 0.0)
    if wo8 != wo_real:
        wmask = jax.lax.broadcasted_iota(
            jnp.int32, (bb, ho, wo8, cout), 2) < wo_real
        out = jnp.where(wmask, out, 0.0)
    o_ref[...] = out.astype(_BF16)


def _conv(xin, w, scale, bias, *, relu, bb, wo_real, res=None):
    n, hi, wi, cin = xin.shape
    cout = w.shape[3]
    ho = hi - 2
    wo8 = wi - 2

    body = functools.partial(_conv_body, wo_real=wo_real, relu=relu,
                             has_res=res is not None)

    full4 = lambda g: (g, 0, 0, 0)
    zero2 = lambda g: (0, 0)
    in_specs = [
        pl.BlockSpec((bb, hi, wi, cin), full4),
        pl.BlockSpec((3, 3, cin, cout), lambda g: (0, 0, 0, 0)),
        pl.BlockSpec((1, cout), zero2),
        pl.BlockSpec((1, cout), zero2),
    ]
    args = [xin, w, scale.reshape(1, cout), bias.reshape(1, cout)]
    if res is not None:
        in_specs.append(pl.BlockSpec((bb, ho, wo8, cout), full4))
        args.append(res)

    outs = pl.pallas_call(
        body,
        grid=(n // bb,),
        in_specs=in_specs,
        out_specs=pl.BlockSpec((bb, ho, wo8, cout), full4),
        out_shape=jax.ShapeDtypeStruct((n, ho, wo8, cout), _BF16),
        compiler_params=pltpu.CompilerParams(
            dimension_semantics=("parallel",),
            vmem_limit_bytes=56 * 1024 * 1024,
        ),
    )(*args)
    return outs


def _conv_s2_body(p00_ref, p01_ref, p10_ref, p11_ref, w_ref, s_ref, b_ref,
                  dw_ref, ds_ref, db_ref, o_ref, o2_ref, *, wo_real):
    bb, hp, wp, cin = p00_ref.shape
    cout = w_ref.shape[3]
    ho = hp - 1
    wo8 = wp - 8
    m = bb * ho * wo8
    phs = [[p00_ref[...], p01_ref[...]], [p10_ref[...], p11_ref[...]]]

    acc = jnp.zeros((m, cout), _F32)
    a_cen = None
    for i in range(3):
        a, dh = i % 2, i // 2
        for j in range(3):
            b, dw = j % 2, j // 2
            win = jax.lax.slice(phs[a][b], (0, dh, dw, 0),
                                (bb, dh + ho, dw + wo8, cin))
            av = win.reshape(m, cin)
            if i == 1 and j == 1:
                a_cen = av
            acc = acc + jnp.dot(av, w_ref[i, j],
                                preferred_element_type=_F32)

    wmask = None
    if wo8 != wo_real:
        wmask = jax.lax.broadcasted_iota(
            jnp.int32, (bb, ho, wo8, cout), 2) < wo_real

    out = acc * s_ref[...] + b_ref[...]
    out = jnp.maximum(out, 0.0).reshape(bb, ho, wo8, cout)
    if wmask is not None:
        out = jnp.where(wmask, out, 0.0)
    o_ref[...] = out.astype(_BF16)

    dout = jnp.dot(a_cen, dw_ref[...], preferred_element_type=_F32)
    dout = (dout * ds_ref[...] + db_ref[...]).reshape(bb, ho, wo8, cout)
    if wmask is not None:
        dout = jnp.where(wmask, dout, 0.0)
    o2_ref[...] = dout.astype(_BF16)


def _conv_s2(y, wo_in, w, scale, bias, down, *, bb):
    n, h, ws, cin = y.shape
    cout = w.shape[3]
    ho = h // 2
    wo = wo_in // 2
    wo8 = _rup(wo, 8)
    wp = wo8 + 8
    need_w = 1 + 2 * (wp - 1) + 1
    xpad = jnp.pad(y, ((0, 0), (1, 1), (1, need_w - ws - 1), (0, 0)))
    phs = []
    for a in range(2):
        for b in range(2):
            ph = xpad[:, a::2, b::2, :][:, :ho + 1, :wp, :]
            ph = jnp.pad(ph, ((0, 0), (0, ho + 1 - ph.shape[1]),
                              (0, wp - ph.shape[2]), (0, 0)))
            phs.append(ph)

    dw, ds, db = down
    full4 = lambda g: (g, 0, 0, 0)
    zero2 = lambda g: (0, 0)
    ph_spec = pl.BlockSpec((bb, ho + 1, wp, cin), full4)
    out_spec = pl.BlockSpec((bb, ho, wo8, cout), full4)
    outs = pl.pallas_call(
        functools.partial(_conv_s2_body, wo_real=wo),
        grid=(n // bb,),
        in_specs=[ph_spec, ph_spec, ph_spec, ph_spec,
                  pl.BlockSpec((3, 3, cin, cout), lambda g: (0, 0, 0, 0)),
                  pl.BlockSpec((1, cout), zero2),
                  pl.BlockSpec((1, cout), zero2),
                  pl.BlockSpec((cin, cout), zero2),
                  pl.BlockSpec((1, cout), zero2),
                  pl.BlockSpec((1, cout), zero2)],
        out_specs=[out_spec, out_spec],
        out_shape=[jax.ShapeDtypeStruct((n, ho, wo8, cout), _BF16),
                   jax.ShapeDtypeStruct((n, ho, wo8, cout), _BF16)],
        compiler_params=pltpu.CompilerParams(
            dimension_semantics=("parallel",),
            vmem_limit_bytes=56 * 1024 * 1024,
        ),
    )(*phs, w, scale.reshape(1, cout), bias.reshape(1, cout),
      dw, ds.reshape(1, cout), db.reshape(1, cout))
    return outs


def _mm_body(a_ref, b_ref, s_ref, c_ref, o_ref, *, relu):
    out = jnp.dot(a_ref[...], b_ref[...], preferred_element_type=_F32)
    out = out * s_ref[...] + c_ref[...]
    if relu:
        out = jnp.maximum(out, 0.0)
    o_ref[...] = out.astype(o_ref.dtype)


def _matmul_bn(a, b, scale, bias, *, relu, tm):
    m, k = a.shape
    cout = b.shape[1]
    outs = pl.pallas_call(
        functools.partial(_mm_body, relu=relu),
        grid=(m // tm,),
        in_specs=[
            pl.BlockSpec((tm, k), lambda g: (g, 0)),
            pl.BlockSpec((k, cout), lambda g: (0, 0)),
            pl.BlockSpec((1, cout), lambda g: (0, 0)),
            pl.BlockSpec((1, cout), lambda g: (0, 0)),
        ],
        out_specs=pl.BlockSpec((tm, cout), lambda g: (g, 0)),
        out_shape=jax.ShapeDtypeStruct((m, cout), _BF16),
        compiler_params=pltpu.CompilerParams(
            dimension_semantics=("parallel",),
            vmem_limit_bytes=56 * 1024 * 1024,
        ),
    )(a, b, scale.reshape(1, cout), bias.reshape(1, cout))
    return outs


def _pool4_body(p00_ref, p01_ref, p10_ref, p11_ref, o_ref):
    p00 = p00_ref[...]
    p01 = p01_ref[...]
    p10 = p10_ref[...]
    p11 = p11_ref[...]
    neg = jnp.array(-jnp.inf, dtype=p00.dtype)

    def shift_w(v):
        return jnp.pad(v[:, :, :-1, :], ((0, 0), (0, 0), (1, 0), (0, 0)),
                       constant_values=neg)

    def shift_h(v):
        return jnp.pad(v[:, :-1, :, :], ((0, 0), (1, 0), (0, 0), (0, 0)),
                       constant_values=neg)

    cm0 = jnp.maximum(p00, jnp.maximum(p01, shift_w(p01)))
    cm1 = jnp.maximum(p10, jnp.maximum(p11, shift_w(p11)))
    o_ref[...] = jnp.maximum(cm0, jnp.maximum(cm1, shift_h(cm1)))


def _maxpool_phases(phases, *, bb):
    n, ho, wo, c = phases[0].shape
    spec = pl.BlockSpec((bb, ho, wo, c), lambda g: (g, 0, 0, 0))
    return pl.pallas_call(
        _pool4_body,
        grid=(n // bb,),
        in_specs=[spec, spec, spec, spec],
        out_specs=spec,
        out_shape=jax.ShapeDtypeStruct((n, ho, wo, c), _BF16),
        compiler_params=pltpu.CompilerParams(
            dimension_semantics=("parallel",),
            vmem_limit_bytes=56 * 1024 * 1024,
        ),
    )(*phases)


def _head_body(x_ref, w_ref, b_ref, o_ref, *, inv_hw):
    feat = jnp.sum(x_ref[...].astype(_F32), axis=1)
    o_ref[...] = (jnp.dot(feat, w_ref[...], preferred_element_type=_F32)
                  * inv_hw + b_ref[...])


def _head(y, fc_w, fc_b, *, real_hw):
    n, hw, c = y.shape
    return pl.pallas_call(
        functools.partial(_head_body, inv_hw=1.0 / real_hw),
        grid=(1,),
        in_specs=[
            pl.BlockSpec((n, hw, c), lambda g: (0, 0, 0)),
            pl.BlockSpec((c, 1), lambda g: (0, 0)),
            pl.BlockSpec((1, 1), lambda g: (0, 0)),
        ],
        out_specs=pl.BlockSpec((n, 1), lambda g: (0, 0)),
        out_shape=jax.ShapeDtypeStruct((n, 1), _F32),
    )(y, fc_w, fc_b.reshape(1, 1))


def _prep_w(w):
    return jnp.transpose(w, (2, 3, 1, 0)).astype(_BF16)


def _basic_block(y, wo_in, p, bb):
    stride = p["stride"]
    n, h, ws, c = y.shape
    wo = wo_in // stride
    wo8 = _rup(wo, 8)

    if "down_w" in p:
        out1, identity = _conv_s2(
            y, wo_in, p["conv1_w"], p["bn1_s"], p["bn1_b"],
            (p["down_w"], p["dbn_s"], p["dbn_b"]), bb=bb)
    else:
        xin = jnp.pad(y, ((0, 0), (1, 1), (1, wo8 + 1 - ws), (0, 0)))
        out1 = _conv(xin, p["conv1_w"], p["bn1_s"], p["bn1_b"],
                     relu=True, bb=bb, wo_real=wo)
        identity = y

    xin2 = jnp.pad(out1, ((0, 0), (1, 1), (1, 1), (0, 0)))
    out2 = _conv(xin2, p["conv2_w"], p["bn2_s"], p["bn2_b"],
                 relu=True, bb=bb, wo_real=wo, res=identity)
    return out2, wo


def kernel(x, conv1_w, bn1_s, bn1_b,
           l0b0_conv1_w, l0b0_bn1_s, l0b0_bn1_b, l0b0_conv2_w, l0b0_bn2_s,
           l0b0_bn2_b,
           l0b1_conv1_w, l0b1_bn1_s, l0b1_bn1_b, l0b1_conv2_w, l0b1_bn2_s,
           l0b1_bn2_b,
           l1b0_conv1_w, l1b0_bn1_s, l1b0_bn1_b, l1b0_conv2_w, l1b0_bn2_s,
           l1b0_bn2_b, l1b0_down_w, l1b0_dbn_s, l1b0_dbn_b,
           l1b1_conv1_w, l1b1_bn1_s, l1b1_bn1_b, l1b1_conv2_w, l1b1_bn2_s,
           l1b1_bn2_b,
           l2b0_conv1_w, l2b0_bn1_s, l2b0_bn1_b, l2b0_conv2_w, l2b0_bn2_s,
           l2b0_bn2_b, l2b0_down_w, l2b0_dbn_s, l2b0_dbn_b,
           l2b1_conv1_w, l2b1_bn1_s, l2b1_bn1_b, l2b1_conv2_w, l2b1_bn2_s,
           l2b1_bn2_b,
           l3b0_conv1_w, l3b0_bn1_s, l3b0_bn1_b, l3b0_conv2_w, l3b0_bn2_s,
           l3b0_bn2_b, l3b0_down_w, l3b0_dbn_s, l3b0_dbn_b,
           l3b1_conv1_w, l3b1_bn1_s, l3b1_bn1_b, l3b1_conv2_w, l3b1_bn2_s,
           l3b1_bn2_b,
           fc_w, fc_b):
    n = x.shape[0]

    xb = x.astype(_BF16)
    pp = []
    for a in range(2):
        row = []
        for b in range(2):
            ph = jnp.transpose(xb[:, :, a::2, b::2], (0, 2, 3, 1))
            row.append(jnp.pad(ph, ((0, 0), (2, 1), (2, 1), (0, 0))))
        pp.append(row)
    wm = jnp.transpose(conv1_w, (2, 3, 1, 0)).reshape(147, 64).astype(_BF16)
    ygrp = []
    for ao in range(2):
        for bo in range(2):
            cols = []
            for i in range(7):
                u = i - 3
                a = u % 2
                dh = (u - a) // 2 + 2 + ao
                for j in range(7):
                    v = j - 3
                    b = v % 2
                    dw = (v - b) // 2 + 2 + bo
                    cols.append(
                        pp[a][b][:, dh:dh + 111:2, dw:dw + 111:2, :])
            patches = jnp.stack(cols, axis=-2).reshape(n * 56 * 56, 147)
            yg = _matmul_bn(patches, wm, bn1_s, bn1_b, relu=True, tm=3136)
            ygrp.append(yg.reshape(n, 56, 56, 64))

    y = _maxpool_phases(ygrp, bb=4)
    wo = 56

    blocks = [
        dict(stride=1, conv1_w=_prep_w(l0b0_conv1_w), bn1_s=l0b0_bn1_s,
             bn1_b=l0b0_bn1_b, conv2_w=_prep_w(l0b0_conv2_w),
             bn2_s=l0b0_bn2_s, bn2_b=l0b0_bn2_b),
        dict(stride=1, conv1_w=_prep_w(l0b1_conv1_w), bn1_s=l0b1_bn1_s,
             bn1_b=l0b1_bn1_b, conv2_w=_prep_w(l0b1_conv2_w),
             bn2_s=l0b1_bn2_s, bn2_b=l0b1_bn2_b),
        dict(stride=2, conv1_w=_prep_w(l1b0_conv1_w), bn1_s=l1b0_bn1_s,
             bn1_b=l1b0_bn1_b, conv2_w=_prep_w(l1b0_conv2_w),
             bn2_s=l1b0_bn2_s, bn2_b=l1b0_bn2_b,
             down_w=jnp.transpose(l1b0_down_w.reshape(128, 64),
                                  (1, 0)).astype(_BF16),
             dbn_s=l1b0_dbn_s, dbn_b=l1b0_dbn_b),
        dict(stride=1, conv1_w=_prep_w(l1b1_conv1_w), bn1_s=l1b1_bn1_s,
             bn1_b=l1b1_bn1_b, conv2_w=_prep_w(l1b1_conv2_w),
             bn2_s=l1b1_bn2_s, bn2_b=l1b1_bn2_b),
        dict(stride=2, conv1_w=_prep_w(l2b0_conv1_w), bn1_s=l2b0_bn1_s,
             bn1_b=l2b0_bn1_b, conv2_w=_prep_w(l2b0_conv2_w),
             bn2_s=l2b0_bn2_s, bn2_b=l2b0_bn2_b,
             down_w=jnp.transpose(l2b0_down_w.reshape(256, 128),
                                  (1, 0)).astype(_BF16),
             dbn_s=l2b0_dbn_s, dbn_b=l2b0_dbn_b),
        dict(stride=1, conv1_w=_prep_w(l2b1_conv1_w), bn1_s=l2b1_bn1_s,
             bn1_b=l2b1_bn1_b, conv2_w=_prep_w(l2b1_conv2_w),
             bn2_s=l2b1_bn2_s, bn2_b=l2b1_bn2_b),
        dict(stride=2, conv1_w=_prep_w(l3b0_conv1_w), bn1_s=l3b0_bn1_s,
             bn1_b=l3b0_bn1_b, conv2_w=_prep_w(l3b0_conv2_w),
             bn2_s=l3b0_bn2_s, bn2_b=l3b0_bn2_b,
             down_w=jnp.transpose(l3b0_down_w.reshape(512, 256),
                                  (1, 0)).astype(_BF16),
             dbn_s=l3b0_dbn_s, dbn_b=l3b0_dbn_b),
        dict(stride=1, conv1_w=_prep_w(l3b1_conv1_w), bn1_s=l3b1_bn1_s,
             bn1_b=l3b1_bn1_b, conv2_w=_prep_w(l3b1_conv2_w),
             bn2_s=l3b1_bn2_s, bn2_b=l3b1_bn2_b),
    ]
    bbs = [1, 1, 2, 2, 8, 8, 16, 16]

    for p, bb in zip(blocks, bbs):
        y, wo = _basic_block(y, wo, p, bb)

    yr = y.reshape(n, y.shape[1] * y.shape[2], y.shape[3])
    return _head(yr, fc_w, fc_b, real_hw=wo * wo)


# speedup vs baseline: 1.4983x; 1.4983x over previous
import functools

import jax
import jax.numpy as jnp
from jax.experimental import pallas as pl
from jax.experimental.pallas import tpu as pltpu

_F32 = jnp.float32
_BF16 = jnp.bfloat16


def _rup(x, m):
    return ((x + m - 1) // m) * m


def _ring_pad(v):
    v = jnp.pad(v, ((0, 0), (0, 0), (1, 1), (0, 0)))
    return jnp.pad(v, ((0, 0), (1, 1), (0, 0), (0, 0)))


def _conv_body(x_ref, w_ref, s_ref, b_ref, *rest, wo_real, relu, has_res):
    i_rest = 0
    if has_res:
        res_ref = rest[i_rest]
        i_rest += 1
    o_ref = rest[i_rest]

    bb, hi, wi, cin = x_ref.shape
    cout = w_ref.shape[3]
    ho = hi - 2
    wo8 = wi - 2
    m = bb * ho * wo8

    x = x_ref[...]
    acc = jnp.zeros((m, cout), _F32)
    for i in range(3):
        for j in range(3):
            win = jax.lax.slice(x, (0, i, j, 0),
                                (bb, i + ho, j + wo8, cin))
            acc = acc + jnp.dot(win.reshape(m, cin), w_ref[i, j],
                                preferred_element_type=_F32)

    out = acc * s_ref[...] + b_ref[...]
    out = out.reshape(bb, ho, wo8, cout)
    if wo8 != wo_real:
        wmask = jax.lax.broadcasted_iota(
            jnp.int32, (bb, ho, wo8, cout), 2) < wo_real
        out = jnp.where(wmask, out, 0.0)
    out = _ring_pad(out)
    if has_res:
        out = out + res_ref[...].astype(_F32)
    if relu:
        out = jnp.maximum(out, 0.0)
    o_ref[...] = out.astype(_BF16)


def _conv(xin, w, scale, bias, *, relu, bb, wo_real, res=None):
    n, hi, wi, cin = xin.shape
    cout = w.shape[3]
    ho = hi - 2
    wo8 = wi - 2

    body = functools.partial(_conv_body, wo_real=wo_real, relu=relu,
                             has_res=res is not None)

    full4 = lambda g: (g, 0, 0, 0)
    zero2 = lambda g: (0, 0)
    in_specs = [
        pl.BlockSpec((bb, hi, wi, cin), full4),
        pl.BlockSpec((3, 3, cin, cout), lambda g: (0, 0, 0, 0)),
        pl.BlockSpec((1, cout), zero2),
        pl.BlockSpec((1, cout), zero2),
    ]
    args = [xin, w, scale.reshape(1, cout), bias.reshape(1, cout)]
    if res is not None:
        in_specs.append(pl.BlockSpec((bb, ho + 2, wo8 + 2, cout), full4))
        args.append(res)

    outs = pl.pallas_call(
        body,
        grid=(n // bb,),
        in_specs=in_specs,
        out_specs=pl.BlockSpec((bb, ho + 2, wo8 + 2, cout), full4),
        out_shape=jax.ShapeDtypeStruct((n, ho + 2, wo8 + 2, cout), _BF16),
        compiler_params=pltpu.CompilerParams(
            dimension_semantics=("parallel",),
            vmem_limit_bytes=56 * 1024 * 1024,
        ),
    )(*args)
    return outs


def _conv_s2_body(p00_ref, p01_ref, p10_ref, p11_ref, w_ref, s_ref, b_ref,
                  dw_ref, ds_ref, db_ref, o_ref, o2_ref, *, wo_real):
    bb, hp, wp, cin = p00_ref.shape
    cout = w_ref.shape[3]
    ho = hp - 1
    wo8 = wp - 8
    m = bb * ho * wo8
    phs = [[p00_ref[...], p01_ref[...]], [p10_ref[...], p11_ref[...]]]

    acc = jnp.zeros((m, cout), _F32)
    a_cen = None
    for i in range(3):
        a, dh = i % 2, i // 2
        for j in range(3):
            b, dw = j % 2, j // 2
            win = jax.lax.slice(phs[a][b], (0, dh, dw, 0),
                                (bb, dh + ho, dw + wo8, cin))
            av = win.reshape(m, cin)
            if i == 1 and j == 1:
                a_cen = av
            acc = acc + jnp.dot(av, w_ref[i, j],
                                preferred_element_type=_F32)

    wmask = None
    if wo8 != wo_real:
        wmask = jax.lax.broadcasted_iota(
            jnp.int32, (bb, ho, wo8, cout), 2) < wo_real

    out = acc * s_ref[...] + b_ref[...]
    out = jnp.maximum(out, 0.0).reshape(bb, ho, wo8, cout)
    if wmask is not None:
        out = jnp.where(wmask, out, 0.0)
    o_ref[...] = _ring_pad(out).astype(_BF16)

    dout = jnp.dot(a_cen, dw_ref[...], preferred_element_type=_F32)
    dout = (dout * ds_ref[...] + db_ref[...]).reshape(bb, ho, wo8, cout)
    if wmask is not None:
        dout = jnp.where(wmask, dout, 0.0)
    o2_ref[...] = _ring_pad(dout).astype(_BF16)


def _conv_s2(y, wo_in, w, scale, bias, down, *, bb):
    n, hp2, wsp2, cin = y.shape
    h = hp2 - 2
    ws = wsp2 - 2
    cout = w.shape[3]
    ho = h // 2
    wo = wo_in // 2
    wo8 = _rup(wo, 8)
    wp = wo8 + 8
    need_w = 1 + 2 * (wp - 1) + 1
    xpad = jnp.pad(y, ((0, 0), (0, 0), (0, need_w - ws - 2), (0, 0)))
    phs = []
    for a in range(2):
        for b in range(2):
            ph = xpad[:, a::2, b::2, :][:, :ho + 1, :wp, :]
            ph = jnp.pad(ph, ((0, 0), (0, ho + 1 - ph.shape[1]),
                              (0, wp - ph.shape[2]), (0, 0)))
            phs.append(ph)

    dw, ds, db = down
    full4 = lambda g: (g, 0, 0, 0)
    zero2 = lambda g: (0, 0)
    ph_spec = pl.BlockSpec((bb, ho + 1, wp, cin), full4)
    out_spec = pl.BlockSpec((bb, ho + 2, wo8 + 2, cout), full4)
    outs = pl.pallas_call(
        functools.partial(_conv_s2_body, wo_real=wo),
        grid=(n // bb,),
        in_specs=[ph_spec, ph_spec, ph_spec, ph_spec,
                  pl.BlockSpec((3, 3, cin, cout), lambda g: (0, 0, 0, 0)),
                  pl.BlockSpec((1, cout), zero2),
                  pl.BlockSpec((1, cout), zero2),
                  pl.BlockSpec((cin, cout), zero2),
                  pl.BlockSpec((1, cout), zero2),
                  pl.BlockSpec((1, cout), zero2)],
        out_specs=[out_spec, out_spec],
        out_shape=[jax.ShapeDtypeStruct((n, ho + 2, wo8 + 2, cout), _BF16),
                   jax.ShapeDtypeStruct((n, ho + 2, wo8 + 2, cout), _BF16)],
        compiler_params=pltpu.CompilerParams(
            dimension_semantics=("parallel",),
            vmem_limit_bytes=56 * 1024 * 1024,
        ),
    )(*phs, w, scale.reshape(1, cout), bias.reshape(1, cout),
      dw, ds.reshape(1, cout), db.reshape(1, cout))
    return outs


def _mm_body(a_ref, b_ref, s_ref, c_ref, o_ref, *, relu):
    out = jnp.dot(a_ref[...], b_ref[...], preferred_element_type=_F32)
    out = out * s_ref[...] + c_ref[...]
    if relu:
        out = jnp.maximum(out, 0.0)
    o_ref[...] = out.astype(o_ref.dtype)


def _matmul_bn(a, b, scale, bias, *, relu, tm):
    m, k = a.shape
    cout = b.shape[1]
    outs = pl.pallas_call(
        functools.partial(_mm_body, relu=relu),
        grid=(m // tm,),
        in_specs=[
            pl.BlockSpec((tm, k), lambda g: (g, 0)),
            pl.BlockSpec((k, cout), lambda g: (0, 0)),
            pl.BlockSpec((1, cout), lambda g: (0, 0)),
            pl.BlockSpec((1, cout), lambda g: (0, 0)),
        ],
        out_specs=pl.BlockSpec((tm, cout), lambda g: (g, 0)),
        out_shape=jax.ShapeDtypeStruct((m, cout), _BF16),
        compiler_params=pltpu.CompilerParams(
            dimension_semantics=("parallel",),
            vmem_limit_bytes=56 * 1024 * 1024,
        ),
    )(a, b, scale.reshape(1, cout), bias.reshape(1, cout))
    return outs


def _pool_body(x_ref, o_ref):
    bb, hi, wi, c = x_ref.shape
    x = x_ref[...]
    neg = jnp.array(-jnp.inf, x.dtype)
    xp = jnp.pad(x, ((0, 0), (0, 0), (1, 1), (0, 0)), constant_values=neg)
    xp = jnp.pad(xp, ((0, 0), (1, 1), (0, 0), (0, 0)), constant_values=neg)
    acc = None
    for i in range(3):
        for j in range(3):
            win = jax.lax.slice(xp, (0, i, j, 0), (bb, i + hi, j + wi, c))
            acc = win if acc is None else jnp.maximum(acc, win)
    ho, wo = hi // 2, wi // 2
    acc = acc.reshape(bb, ho, 2, wi, c)[:, :, 0, :, :]
    acc = acc.reshape(bb, ho, wo, 2, c)[:, :, :, 0, :]
    o_ref[...] = _ring_pad(acc).astype(_BF16)


def _maxpool(x, *, bb):
    n, hi, wi, c = x.shape
    ho, wo = hi // 2, wi // 2
    return pl.pallas_call(
        _pool_body,
        grid=(n // bb,),
        in_specs=[pl.BlockSpec((bb, hi, wi, c), lambda g: (g, 0, 0, 0))],
        out_specs=pl.BlockSpec((bb, ho + 2, wo + 2, c),
                               lambda g: (g, 0, 0, 0)),
        out_shape=jax.ShapeDtypeStruct((n, ho + 2, wo + 2, c), _BF16),
        compiler_params=pltpu.CompilerParams(
            dimension_semantics=("parallel",),
            vmem_limit_bytes=56 * 1024 * 1024,
        ),
    )(x)


def _head_body(x_ref, w_ref, b_ref, o_ref, *, inv_hw):
    feat = jnp.sum(x_ref[...].astype(_F32), axis=1)
    o_ref[...] = (jnp.dot(feat, w_ref[...], preferred_element_type=_F32)
                  * inv_hw + b_ref[...])


def _head(y, fc_w, fc_b, *, real_hw):
    n, hw, c = y.shape
    return pl.pallas_call(
        functools.partial(_head_body, inv_hw=1.0 / real_hw),
        grid=(1,),
        in_specs=[
            pl.BlockSpec((n, hw, c), lambda g: (0, 0, 0)),
            pl.BlockSpec((c, 1), lambda g: (0, 0)),
            pl.BlockSpec((1, 1), lambda g: (0, 0)),
        ],
        out_specs=pl.BlockSpec((n, 1), lambda g: (0, 0)),
        out_shape=jax.ShapeDtypeStruct((n, 1), _F32),
    )(y, fc_w, fc_b.reshape(1, 1))


def _prep_w(w):
    return jnp.transpose(w, (2, 3, 1, 0)).astype(_BF16)


def _basic_block(y, wo_in, p, bb):
    stride = p["stride"]
    wo = wo_in // stride

    if "down_w" in p:
        out1, identity = _conv_s2(
            y, wo_in, p["conv1_w"], p["bn1_s"], p["bn1_b"],
            (p["down_w"], p["dbn_s"], p["dbn_b"]), bb=bb)
    else:
        out1 = _conv(y, p["conv1_w"], p["bn1_s"], p["bn1_b"],
                     relu=True, bb=bb, wo_real=wo)
        identity = y

    out2 = _conv(out1, p["conv2_w"], p["bn2_s"], p["bn2_b"],
                 relu=True, bb=bb, wo_real=wo, res=identity)
    return out2, wo


def kernel(x, conv1_w, bn1_s, bn1_b,
           l0b0_conv1_w, l0b0_bn1_s, l0b0_bn1_b, l0b0_conv2_w, l0b0_bn2_s,
           l0b0_bn2_b,
           l0b1_conv1_w, l0b1_bn1_s, l0b1_bn1_b, l0b1_conv2_w, l0b1_bn2_s,
           l0b1_bn2_b,
           l1b0_conv1_w, l1b0_bn1_s, l1b0_bn1_b, l1b0_conv2_w, l1b0_bn2_s,
           l1b0_bn2_b, l1b0_down_w, l1b0_dbn_s, l1b0_dbn_b,
           l1b1_conv1_w, l1b1_bn1_s, l1b1_bn1_b, l1b1_conv2_w, l1b1_bn2_s,
           l1b1_bn2_b,
           l2b0_conv1_w, l2b0_bn1_s, l2b0_bn1_b, l2b0_conv2_w, l2b0_bn2_s,
           l2b0_bn2_b, l2b0_down_w, l2b0_dbn_s, l2b0_dbn_b,
           l2b1_conv1_w, l2b1_bn1_s, l2b1_bn1_b, l2b1_conv2_w, l2b1_bn2_s,
           l2b1_bn2_b,
           l3b0_conv1_w, l3b0_bn1_s, l3b0_bn1_b, l3b0_conv2_w, l3b0_bn2_s,
           l3b0_bn2_b, l3b0_down_w, l3b0_dbn_s, l3b0_dbn_b,
           l3b1_conv1_w, l3b1_bn1_s, l3b1_bn1_b, l3b1_conv2_w, l3b1_bn2_s,
           l3b1_bn2_b,
           fc_w, fc_b):
    n = x.shape[0]

    xb = x.astype(_BF16)
    pp = []
    for a in range(2):
        row = []
        for b in range(2):
            ph = jnp.transpose(xb[:, :, a::2, b::2], (0, 2, 3, 1))
            row.append(jnp.pad(ph, ((0, 0), (2, 1), (2, 1), (0, 0))))
        pp.append(row)
    cols = []
    for i in range(7):
        u = i - 3
        a = u % 2
        dh = (u - a) // 2 + 2
        for j in range(7):
            v = j - 3
            b = v % 2
            dw = (v - b) // 2 + 2
            cols.append(pp[a][b][:, dh:dh + 112, dw:dw + 112, :])
    patches = jnp.stack(cols, axis=-2).reshape(n * 112 * 112, 147)
    wm = jnp.transpose(conv1_w, (2, 3, 1, 0)).reshape(147, 64).astype(_BF16)
    y = _matmul_bn(patches, wm, bn1_s, bn1_b, relu=True, tm=2048)
    y = y.reshape(n, 112, 112, 64)

    y = _maxpool(y, bb=2)
    wo = 56

    blocks = [
        dict(stride=1, conv1_w=_prep_w(l0b0_conv1_w), bn1_s=l0b0_bn1_s,
             bn1_b=l0b0_bn1_b, conv2_w=_prep_w(l0b0_conv2_w),
             bn2_s=l0b0_bn2_s, bn2_b=l0b0_bn2_b),
        dict(stride=1, conv1_w=_prep_w(l0b1_conv1_w), bn1_s=l0b1_bn1_s,
             bn1_b=l0b1_bn1_b, conv2_w=_prep_w(l0b1_conv2_w),
             bn2_s=l0b1_bn2_s, bn2_b=l0b1_bn2_b),
        dict(stride=2, conv1_w=_prep_w(l1b0_conv1_w), bn1_s=l1b0_bn1_s,
             bn1_b=l1b0_bn1_b, conv2_w=_prep_w(l1b0_conv2_w),
             bn2_s=l1b0_bn2_s, bn2_b=l1b0_bn2_b,
             down_w=jnp.transpose(l1b0_down_w.reshape(128, 64),
                                  (1, 0)).astype(_BF16),
             dbn_s=l1b0_dbn_s, dbn_b=l1b0_dbn_b),
        dict(stride=1, conv1_w=_prep_w(l1b1_conv1_w), bn1_s=l1b1_bn1_s,
             bn1_b=l1b1_bn1_b, conv2_w=_prep_w(l1b1_conv2_w),
             bn2_s=l1b1_bn2_s, bn2_b=l1b1_bn2_b),
        dict(stride=2, conv1_w=_prep_w(l2b0_conv1_w), bn1_s=l2b0_bn1_s,
             bn1_b=l2b0_bn1_b, conv2_w=_prep_w(l2b0_conv2_w),
             bn2_s=l2b0_bn2_s, bn2_b=l2b0_bn2_b,
             down_w=jnp.transpose(l2b0_down_w.reshape(256, 128),
                                  (1, 0)).astype(_BF16),
             dbn_s=l2b0_dbn_s, dbn_b=l2b0_dbn_b),
        dict(stride=1, conv1_w=_prep_w(l2b1_conv1_w), bn1_s=l2b1_bn1_s,
             bn1_b=l2b1_bn1_b, conv2_w=_prep_w(l2b1_conv2_w),
             bn2_s=l2b1_bn2_s, bn2_b=l2b1_bn2_b),
        dict(stride=2, conv1_w=_prep_w(l3b0_conv1_w), bn1_s=l3b0_bn1_s,
             bn1_b=l3b0_bn1_b, conv2_w=_prep_w(l3b0_conv2_w),
             bn2_s=l3b0_bn2_s, bn2_b=l3b0_bn2_b,
             down_w=jnp.transpose(l3b0_down_w.reshape(512, 256),
                                  (1, 0)).astype(_BF16),
             dbn_s=l3b0_dbn_s, dbn_b=l3b0_dbn_b),
        dict(stride=1, conv1_w=_prep_w(l3b1_conv1_w), bn1_s=l3b1_bn1_s,
             bn1_b=l3b1_bn1_b, conv2_w=_prep_w(l3b1_conv2_w),
             bn2_s=l3b1_bn2_s, bn2_b=l3b1_bn2_b),
    ]
    bbs = [1, 1, 2, 2, 8, 8, 16, 16]

    for p, bb in zip(blocks, bbs):
        y, wo = _basic_block(y, wo, p, bb)

    yr = y[:, 1:-1, 1:-1, :]
    yr = yr.reshape(n, yr.shape[1] * yr.shape[2], yr.shape[3])
    return _head(yr, fc_w, fc_b, real_hw=wo * wo)
